# baseline (speedup 1.0000x reference)
import functools

import jax
import jax.numpy as jnp
from jax.experimental import pallas as pl

F32 = jnp.float32
BF16 = jnp.bfloat16


def _deg_body(nsteps, f_ref, t_ref, i_ref,
              df_ref, dt_ref, di_ref, rs_ref, ft_ref, tt_ref, it_ref):
    k = pl.program_id(0)
    f = f_ref[...]
    t = t_ref[...]
    i = i_ref[...]
    ft_ref[...] = f.astype(BF16).T
    tt_ref[...] = t.astype(BF16).T
    it_ref[...] = i.astype(BF16).T
    rs_ref[...] = jnp.sum(t, axis=1, keepdims=True)

    @pl.when(k == 0)
    def _():
        df_ref[...] = jnp.zeros_like(df_ref)
        dt_ref[...] = jnp.zeros_like(dt_ref)
        di_ref[...] = jnp.zeros_like(di_ref)

    df_ref[...] += jnp.sum(f, axis=0, keepdims=True)
    dt_ref[...] += jnp.sum(t, axis=0, keepdims=True)
    di_ref[...] += jnp.sum(i, axis=0, keepdims=True)

    @pl.when(k == nsteps - 1)
    def _():
        for r in (df_ref, dt_ref, di_ref):
            s = r[...]
            r[...] = jnp.where(s > 0, jax.lax.rsqrt(s), 0.0)


def _degrees(adj_f, adj_t, adj_i, bk=256):
    n = adj_f.shape[0]
    nk = n // bk
    return pl.pallas_call(
        functools.partial(_deg_body, nk),
        grid=(nk,),
        in_specs=[pl.BlockSpec((bk, n), lambda k: (k, 0))] * 3,
        out_specs=[pl.BlockSpec((1, n), lambda k: (0, 0))] * 3
        + [pl.BlockSpec((bk, 1), lambda k: (k, 0))]
        + [pl.BlockSpec((n, bk), lambda k: (0, k))] * 3,
        out_shape=[jax.ShapeDtypeStruct((1, n), F32)] * 3
        + [jax.ShapeDtypeStruct((n, 1), F32)]
        + [jax.ShapeDtypeStruct((n, n), BF16)] * 3,
    )(adj_f, adj_t, adj_i)


def _mmT_body(a_ref, y_ref, dis_ref, b_ref, o_ref):
    acc = jnp.dot(a_ref[...], y_ref[...], preferred_element_type=F32)
    o_ref[...] = acc * dis_ref[...] + b_ref[...]


def _gcn_mmT(AT, Y, dis_col, bias_row, bi=1024):
    n = AT.shape[0]
    w = Y.shape[1]
    return pl.pallas_call(
        _mmT_body,
        grid=(n // bi,),
        in_specs=[
            pl.BlockSpec((bi, n), lambda i: (i, 0)),
            pl.BlockSpec((n, w), lambda i: (0, 0)),
            pl.BlockSpec((bi, 1), lambda i: (i, 0)),
            pl.BlockSpec((1, w), lambda i: (0, 0)),
        ],
        out_specs=pl.BlockSpec((bi, w), lambda i: (i, 0)),
        out_shape=jax.ShapeDtypeStruct((n, w), F32),
    )(AT, Y, dis_col, bias_row)


def _prep1_body(feat_ref, feata_ref, w1_ref, df_ref, dt_ref, di_ref,
                yf_ref, yt_ref, ya_ref, yi_ref):
    w1 = w1_ref[...]
    p = jnp.dot(feat_ref[...], w1, preferred_element_type=F32)
    pa = jnp.dot(feata_ref[...], w1, preferred_element_type=F32)
    yf_ref[...] = (df_ref[...] * p).astype(BF16)
    yt_ref[...] = (dt_ref[...] * p).astype(BF16)
    ya_ref[...] = (dt_ref[...] * pa).astype(BF16)
    yi_ref[...] = (di_ref[...] * p).astype(BF16)


def _bn_relu(h, gamma, beta):
    m = jnp.mean(h, axis=0, keepdims=True)
    c = h - m
    v = jnp.mean(c * c, axis=0, keepdims=True)
    return jnp.maximum(c * jax.lax.rsqrt(v + 1e-5) * gamma + beta, 0.0)


def _mid_body(hf_ref, hta_ref, hi_ref, g_ref, b_ref, w2_ref,
              df_ref, dt_ref, di_ref, yf_ref, yt_ref, ya_ref, yi_ref):
    g = g_ref[...]
    b = b_ref[...]
    w2 = w2_ref[...]
    dh = w2.shape[0]

    def stage(h, dis):
        return (dis * jnp.dot(_bn_relu(h, g, b), w2,
                              preferred_element_type=F32)).astype(BF16)

    yf_ref[...] = stage(hf_ref[...], df_ref[...])
    yt_ref[...] = stage(hta_ref[:, :dh], dt_ref[...])
    ya_ref[...] = stage(hta_ref[:, dh:], dt_ref[...])
    yi_ref[...] = stage(hi_ref[...], di_ref[...])


def _readout_body(tt_ref, y_ref, rs_ref, o_ref):
    acc = jax.lax.dot_general(
        tt_ref[...], y_ref[...].astype(BF16), (((0,), (0,)), ((), ())),
        preferred_element_type=F32)
    gg = acc / rs_ref[...]
    nrm = jnp.sqrt(jnp.sum(gg * gg, axis=1, keepdims=True))
    gg = gg / jnp.maximum(nrm, 1e-12)
    o_ref[...] = jax.nn.sigmoid(gg)


def _readout(adj_tT, h1_t, rs_t, bi=1024):
    n = adj_tT.shape[0]
    w = h1_t.shape[1]
    return pl.pallas_call(
        _readout_body,
        grid=(n // bi,),
        in_specs=[
            pl.BlockSpec((n, bi), lambda i: (0, i)),
            pl.BlockSpec((n, w), lambda i: (0, 0)),
            pl.BlockSpec((bi, 1), lambda i: (i, 0)),
        ],
        out_specs=pl.BlockSpec((bi, w), lambda i: (i, 0)),
        out_shape=jax.ShapeDtypeStruct((n, w), F32),
    )(adj_tT, h1_t, rs_t)


def _att_body(hf_ref, ht_ref, ha_ref, hi_ref, g_ref, wb_ref, bb_ref,
              wa1_ref, ba1_ref, wa2_ref,
              sc1_ref, sc2_ref, hid_ref, bf_ref, bt_ref, bi_ref):
    hf = hf_ref[...]
    ht = ht_ref[...]
    ha = ha_ref[...]
    hi = hi_ref[...]
    g = g_ref[...]
    t = jax.lax.dot_general(
        g, wb_ref[...], (((1,), (1,)), ((), ())), preferred_element_type=F32)
    sc1_ref[...] = jnp.sum(ht * t, axis=1, keepdims=True) + bb_ref[...]
    sc2_ref[...] = jnp.sum(ha * t, axis=1, keepdims=True) + bb_ref[...]

    wa1 = wa1_ref[...]
    ba1 = ba1_ref[...]
    wa2 = wa2_ref[...]

    def score(h):
        e = jnp.tanh(jnp.dot(h, wa1, preferred_element_type=F32) + ba1)
        return jnp.sum(e * wa2, axis=1, keepdims=True)

    ef = score(hf)
    et = score(ht)
    ei = score(hi)
    mx = jnp.maximum(jnp.maximum(ef, et), ei)
    xf = jnp.exp(ef - mx)
    xt = jnp.exp(et - mx)
    xi = jnp.exp(ei - mx)
    s = xf + xt + xi
    bf = xf / s
    bt = xt / s
    bi = xi / s
    bf_ref[...] = bf
    bt_ref[...] = bt
    bi_ref[...] = bi
    hid_ref[...] = bf * hf + bt * ht + bi * hi


def _wadj_body(nsteps, ft_ref, tt_ref, it_ref, af_ref, at_ref, ai_ref,
               a_ref, c_ref, r_ref):
    k = pl.program_id(0)
    blk = (af_ref[...] * ft_ref[...].astype(F32)
           + at_ref[...] * tt_ref[...].astype(F32)
           + ai_ref[...] * it_ref[...].astype(F32))
    a_ref[...] = blk.astype(BF16)
    c_ref[...] = jnp.sum(blk, axis=1, keepdims=True)

    @pl.when(k == 0)
    def _():
        r_ref[...] = jnp.zeros_like(r_ref)

    r_ref[...] += jnp.sum(blk, axis=0, keepdims=True)


def _weighted_adj(adj_fT, adj_tT, adj_iT, af_row, at_row, ai_row, bk=256):
    n = adj_fT.shape[0]
    nk = n // bk
    return pl.pallas_call(
        functools.partial(_wadj_body, nk),
        grid=(nk,),
        in_specs=[pl.BlockSpec((bk, n), lambda k: (k, 0))] * 3
        + [pl.BlockSpec((1, n), lambda k: (0, 0))] * 3,
        out_specs=[
            pl.BlockSpec((bk, n), lambda k: (k, 0)),
            pl.BlockSpec((bk, 1), lambda k: (k, 0)),
            pl.BlockSpec((1, n), lambda k: (0, 0)),
        ],
        out_shape=[
            jax.ShapeDtypeStruct((n, n), BF16),
            jax.ShapeDtypeStruct((n, 1), F32),
            jax.ShapeDtypeStruct((1, n), F32),
        ],
    )(adj_fT, adj_tT, adj_iT, af_row, at_row, ai_row)


def _dec_in_body(cs_ref, rs_ref, hid_ref, w3_ref, y3_ref, dis_ref):
    dw = (cs_ref[...] + rs_ref[...]) / 3.0
    dis = jnp.where(dw > 0, jax.lax.rsqrt(dw), 0.0)
    dis_ref[...] = dis
    y3_ref[...] = (dis * jnp.dot(hid_ref[...], w3_ref[...],
                                 preferred_element_type=F32)).astype(BF16)


def _sym_body(relu, a1_ref, a2_ref, y_ref, dis_ref, b_ref, o_ref):
    y = y_ref[...]
    acc = jnp.dot(a1_ref[...], y, preferred_element_type=F32)
    acc += jax.lax.dot_general(
        a2_ref[...], y, (((0,), (0,)), ((), ())), preferred_element_type=F32)
    r = dis_ref[...] * (acc * (1.0 / 3.0)) + b_ref[...]
    if relu:
        r = jnp.maximum(r, 0.0)
    o_ref[...] = r


def _sym_mm(adj123T, Y, dis_col, bias_row, relu, bi=1024):
    n = adj123T.shape[0]
    w = Y.shape[1]
    return pl.pallas_call(
        functools.partial(_sym_body, relu),
        grid=(n // bi,),
        in_specs=[
            pl.BlockSpec((bi, n), lambda i: (i, 0)),
            pl.BlockSpec((n, bi), lambda i: (0, i)),
            pl.BlockSpec((n, w), lambda i: (0, 0)),
            pl.BlockSpec((bi, 1), lambda i: (i, 0)),
            pl.BlockSpec((1, w), lambda i: (0, 0)),
        ],
        out_specs=pl.BlockSpec((bi, w), lambda i: (i, 0)),
        out_shape=jax.ShapeDtypeStruct((n, w), F32),
    )(adj123T, adj123T, Y, dis_col, bias_row)


def _dec_mid_body(h_ref, g_ref, b_ref, w4_ref, dis_ref, y4_ref):
    y4_ref[...] = (dis_ref[...] * jnp.dot(
        _bn_relu(h_ref[...], g_ref[...], b_ref[...]), w4_ref[...],
        preferred_element_type=F32)).astype(BF16)


def _simple_call(body, out_shapes, *args):
    return pl.pallas_call(body, out_shape=out_shapes)(*args)


def kernel(feat, feat_a, adj_f, adj_t, adj_i, W1, b1, W2, b2, W3, b3, W4, b4,
           bn_gamma, bn_beta, Wa1, ba1, Wa2, Wb, bb):
    n = adj_f.shape[0]
    din = W1.shape[0]
    dh = W1.shape[1]
    dout = W2.shape[1]

    gamma = bn_gamma.reshape(1, dh)
    beta = bn_beta.reshape(1, dh)
    b1r = b1.reshape(1, dh)
    b2r = b2.reshape(1, dout)
    b3r = b3.reshape(1, dh)
    b4r = b4.reshape(1, din)
    ba1r = ba1.reshape(1, -1)
    wa2r = Wa2.reshape(1, -1)
    bbr = bb.reshape(1, 1)
    wb0 = Wb[0]

    dis_f, dis_t, dis_i, rs_t, adj_fT, adj_tT, adj_iT = _degrees(
        adj_f, adj_t, adj_i)
    dis_f = dis_f.reshape(n, 1)
    dis_t = dis_t.reshape(n, 1)
    dis_i = dis_i.reshape(n, 1)

    sds = jax.ShapeDtypeStruct((n, dh), BF16)
    y1_f, y1_t, y1_a, y1_i = _simple_call(
        _prep1_body, [sds, sds, sds, sds],
        feat, feat_a, W1, dis_f, dis_t, dis_i)
    y1_ta = jnp.concatenate([y1_t, y1_a], axis=1)

    h1a_f = _gcn_mmT(adj_fT, y1_f, dis_f, b1r)
    h1a_ta = _gcn_mmT(adj_tT, y1_ta, dis_t, jnp.concatenate([b1r, b1r], axis=1))
    h1a_i = _gcn_mmT(adj_iT, y1_i, dis_i, b1r)

    sds2 = jax.ShapeDtypeStruct((n, dout), BF16)
    y2_f, y2_t, y2_a, y2_i = _simple_call(
        _mid_body, [sds2, sds2, sds2, sds2],
        h1a_f, h1a_ta, h1a_i, gamma, beta, W2, dis_f, dis_t, dis_i)
    y2_ta = jnp.concatenate([y2_t, y2_a], axis=1)

    h1_f = _gcn_mmT(adj_fT, y2_f, dis_f, b2r)
    h1_ta = _gcn_mmT(adj_tT, y2_ta, dis_t, jnp.concatenate([b2r, b2r], axis=1))
    h1_i = _gcn_mmT(adj_iT, y2_i, dis_i, b2r)
    h1_t = h1_ta[:, :dout]
    h1_a = h1_ta[:, dout:]

    g = _readout(adj_tT, h1_t, rs_t)

    col = jax.ShapeDtypeStruct((n, 1), F32)
    sc1, sc2, hiden_emb, att_f, att_t, att_i = _simple_call(
        _att_body,
        [col, col, jax.ShapeDtypeStruct((n, dout), F32), col, col, col],
        h1_f, h1_t, h1_a, h1_i, g, wb0, bbr, Wa1, ba1r, wa2r)
    ret = jnp.concatenate([sc1, sc2], axis=1)

    adj123T, cs123, rs123 = _weighted_adj(
        adj_fT, adj_tT, adj_iT,
        att_f.reshape(1, n), att_t.reshape(1, n), att_i.reshape(1, n))

    y3, dis_w = _simple_call(
        _dec_in_body,
        [jax.ShapeDtypeStruct((n, dh), BF16),
         jax.ShapeDtypeStruct((n, 1), F32)],
        cs123, rs123.reshape(n, 1), hiden_emb, W3)

    h2a = _sym_mm(adj123T, y3, dis_w, b3r, relu=False)

    y4 = _simple_call(
        _dec_mid_body, jax.ShapeDtypeStruct((n, din), BF16),
        h2a, gamma, beta, W4, dis_w)

    h2 = _sym_mm(adj123T, y4, dis_w, b4r, relu=True)

    return (hiden_emb, h2, ret)


# speedup vs baseline: 1.0063x; 1.0063x over previous
import functools

import jax
import jax.numpy as jnp
from jax.experimental import pallas as pl

F32 = jnp.float32
BF16 = jnp.bfloat16


def _deg_body(nsteps, f_ref, t_ref, i_ref,
              df_ref, dt_ref, di_ref, rs_ref, ft_ref, tt_ref, it_ref):
    k = pl.program_id(0)
    f = f_ref[...]
    t = t_ref[...]
    i = i_ref[...]
    ft_ref[...] = f.astype(BF16)
    tt_ref[...] = t.astype(BF16)
    it_ref[...] = i.astype(BF16)
    rs_ref[...] = jnp.sum(t, axis=1, keepdims=True)

    @pl.when(k == 0)
    def _():
        df_ref[...] = jnp.zeros_like(df_ref)
        dt_ref[...] = jnp.zeros_like(dt_ref)
        di_ref[...] = jnp.zeros_like(di_ref)

    df_ref[...] += jnp.sum(f, axis=0, keepdims=True)
    dt_ref[...] += jnp.sum(t, axis=0, keepdims=True)
    di_ref[...] += jnp.sum(i, axis=0, keepdims=True)

    @pl.when(k == nsteps - 1)
    def _():
        for r in (df_ref, dt_ref, di_ref):
            s = r[...]
            r[...] = jnp.where(s > 0, jax.lax.rsqrt(s), 0.0)


def _degrees(adj_f, adj_t, adj_i, bk=256):
    n = adj_f.shape[0]
    nk = n // bk
    return pl.pallas_call(
        functools.partial(_deg_body, nk),
        grid=(nk,),
        in_specs=[pl.BlockSpec((bk, n), lambda k: (k, 0))] * 3,
        out_specs=[pl.BlockSpec((1, n), lambda k: (0, 0))] * 3
        + [pl.BlockSpec((bk, 1), lambda k: (k, 0))]
        + [pl.BlockSpec((bk, n), lambda k: (k, 0))] * 3,
        out_shape=[jax.ShapeDtypeStruct((1, n), F32)] * 3
        + [jax.ShapeDtypeStruct((n, 1), F32)]
        + [jax.ShapeDtypeStruct((n, n), BF16)] * 3,
    )(adj_f, adj_t, adj_i)


def _mmT_body(a_ref, y_ref, dis_ref, b_ref, o_ref):
    acc = jax.lax.dot_general(
        a_ref[...], y_ref[...], (((0,), (0,)), ((), ())),
        preferred_element_type=F32)
    o_ref[...] = acc * dis_ref[...] + b_ref[...]


def _gcn_mmT(AT, Y, dis_col, bias_row, bi=1024):
    n = AT.shape[0]
    w = Y.shape[1]
    return pl.pallas_call(
        _mmT_body,
        grid=(n // bi,),
        in_specs=[
            pl.BlockSpec((n, bi), lambda i: (0, i)),
            pl.BlockSpec((n, w), lambda i: (0, 0)),
            pl.BlockSpec((bi, 1), lambda i: (i, 0)),
            pl.BlockSpec((1, w), lambda i: (0, 0)),
        ],
        out_specs=pl.BlockSpec((bi, w), lambda i: (i, 0)),
        out_shape=jax.ShapeDtypeStruct((n, w), F32),
    )(AT, Y, dis_col, bias_row)


def _prep1_body(feat_ref, feata_ref, w1_ref, df_ref, dt_ref, di_ref,
                yf_ref, yt_ref, ya_ref, yi_ref):
    w1 = w1_ref[...]
    p = jnp.dot(feat_ref[...], w1, preferred_element_type=F32)
    pa = jnp.dot(feata_ref[...], w1, preferred_element_type=F32)
    yf_ref[...] = (df_ref[...] * p).astype(BF16)
    yt_ref[...] = (dt_ref[...] * p).astype(BF16)
    ya_ref[...] = (dt_ref[...] * pa).astype(BF16)
    yi_ref[...] = (di_ref[...] * p).astype(BF16)


def _bn_relu(h, gamma, beta):
    m = jnp.mean(h, axis=0, keepdims=True)
    c = h - m
    v = jnp.mean(c * c, axis=0, keepdims=True)
    return jnp.maximum(c * jax.lax.rsqrt(v + 1e-5) * gamma + beta, 0.0)


def _mid_body(hf_ref, hta_ref, hi_ref, g_ref, b_ref, w2_ref,
              df_ref, dt_ref, di_ref, yf_ref, yt_ref, ya_ref, yi_ref):
    g = g_ref[...]
    b = b_ref[...]
    w2 = w2_ref[...]
    dh = w2.shape[0]

    def stage(h, dis):
        return (dis * jnp.dot(_bn_relu(h, g, b), w2,
                              preferred_element_type=F32)).astype(BF16)

    yf_ref[...] = stage(hf_ref[...], df_ref[...])
    yt_ref[...] = stage(hta_ref[:, :dh], dt_ref[...])
    ya_ref[...] = stage(hta_ref[:, dh:], dt_ref[...])
    yi_ref[...] = stage(hi_ref[...], di_ref[...])


def _readout_body(tt_ref, y_ref, rs_ref, o_ref):
    acc = jnp.dot(tt_ref[...], y_ref[...].astype(BF16),
                  preferred_element_type=F32)
    gg = acc / rs_ref[...]
    nrm = jnp.sqrt(jnp.sum(gg * gg, axis=1, keepdims=True))
    gg = gg / jnp.maximum(nrm, 1e-12)
    o_ref[...] = jax.nn.sigmoid(gg)


def _readout(adj_tT, h1_t, rs_t, bi=1024):
    n = adj_tT.shape[0]
    w = h1_t.shape[1]
    return pl.pallas_call(
        _readout_body,
        grid=(n // bi,),
        in_specs=[
            pl.BlockSpec((bi, n), lambda i: (i, 0)),
            pl.BlockSpec((n, w), lambda i: (0, 0)),
            pl.BlockSpec((bi, 1), lambda i: (i, 0)),
        ],
        out_specs=pl.BlockSpec((bi, w), lambda i: (i, 0)),
        out_shape=jax.ShapeDtypeStruct((n, w), F32),
    )(adj_tT, h1_t, rs_t)


def _att_body(hf_ref, ht_ref, ha_ref, hi_ref, g_ref, wb_ref, bb_ref,
              wa1_ref, ba1_ref, wa2_ref,
              sc1_ref, sc2_ref, hid_ref, bf_ref, bt_ref, bi_ref):
    hf = hf_ref[...]
    ht = ht_ref[...]
    ha = ha_ref[...]
    hi = hi_ref[...]
    g = g_ref[...]
    wb = wb_ref[...]
    u_t = jnp.dot(ht, wb, preferred_element_type=F32)
    u_a = jnp.dot(ha, wb, preferred_element_type=F32)
    sc1_ref[...] = jnp.sum(u_t * g, axis=1, keepdims=True) + bb_ref[...]
    sc2_ref[...] = jnp.sum(u_a * g, axis=1, keepdims=True) + bb_ref[...]

    wa1 = wa1_ref[...]
    ba1 = ba1_ref[...]
    wa2 = wa2_ref[...]

    def score(h):
        e = jnp.tanh(jnp.dot(h, wa1, preferred_element_type=F32) + ba1)
        return jnp.sum(e * wa2, axis=1, keepdims=True)

    ef = score(hf)
    et = score(ht)
    ei = score(hi)
    mx = jnp.maximum(jnp.maximum(ef, et), ei)
    xf = jnp.exp(ef - mx)
    xt = jnp.exp(et - mx)
    xi = jnp.exp(ei - mx)
    s = xf + xt + xi
    bf = xf / s
    bt = xt / s
    bi = xi / s
    bf_ref[...] = bf
    bt_ref[...] = bt
    bi_ref[...] = bi
    hid_ref[...] = bf * hf + bt * ht + bi * hi


def _wadj_body(nsteps, ft_ref, tt_ref, it_ref, af_ref, at_ref, ai_ref,
               a_ref, c_ref, r_ref):
    k = pl.program_id(0)
    blk = (af_ref[...] * ft_ref[...].astype(F32)
           + at_ref[...] * tt_ref[...].astype(F32)
           + ai_ref[...] * it_ref[...].astype(F32))
    a_ref[...] = blk.astype(BF16)
    r_ref[...] = jnp.sum(blk, axis=1, keepdims=True)

    @pl.when(k == 0)
    def _():
        c_ref[...] = jnp.zeros_like(c_ref)

    c_ref[...] += jnp.sum(blk, axis=0, keepdims=True)


def _weighted_adj(adj_fT, adj_tT, adj_iT, af_row, at_row, ai_row, bk=256):
    n = adj_fT.shape[0]
    nk = n // bk
    return pl.pallas_call(
        functools.partial(_wadj_body, nk),
        grid=(nk,),
        in_specs=[pl.BlockSpec((bk, n), lambda k: (k, 0))] * 3
        + [pl.BlockSpec((bk, 1), lambda k: (k, 0))] * 3,
        out_specs=[
            pl.BlockSpec((bk, n), lambda k: (k, 0)),
            pl.BlockSpec((1, n), lambda k: (0, 0)),
            pl.BlockSpec((bk, 1), lambda k: (k, 0)),
        ],
        out_shape=[
            jax.ShapeDtypeStruct((n, n), BF16),
            jax.ShapeDtypeStruct((1, n), F32),
            jax.ShapeDtypeStruct((n, 1), F32),
        ],
    )(adj_fT, adj_tT, adj_iT, af_row, at_row, ai_row)


def _dec_in_body(cs_ref, rs_ref, hid_ref, w3_ref, y3_ref, dis_ref):
    dw = (cs_ref[...] + rs_ref[...]) / 3.0
    dis = jnp.where(dw > 0, jax.lax.rsqrt(dw), 0.0)
    dis_ref[...] = dis
    y3_ref[...] = (dis * jnp.dot(hid_ref[...], w3_ref[...],
                                 preferred_element_type=F32)).astype(BF16)


def _sym_body(relu, a1_ref, a2_ref, y_ref, dis_ref, b_ref, o_ref):
    y = y_ref[...]
    acc = jnp.dot(a1_ref[...], y, preferred_element_type=F32)
    acc += jax.lax.dot_general(
        a2_ref[...], y, (((0,), (0,)), ((), ())), preferred_element_type=F32)
    r = dis_ref[...] * (acc * (1.0 / 3.0)) + b_ref[...]
    if relu:
        r = jnp.maximum(r, 0.0)
    o_ref[...] = r


def _sym_mm(adj123T, Y, dis_col, bias_row, relu, bi=1024):
    n = adj123T.shape[0]
    w = Y.shape[1]
    return pl.pallas_call(
        functools.partial(_sym_body, relu),
        grid=(n // bi,),
        in_specs=[
            pl.BlockSpec((bi, n), lambda i: (i, 0)),
            pl.BlockSpec((n, bi), lambda i: (0, i)),
            pl.BlockSpec((n, w), lambda i: (0, 0)),
            pl.BlockSpec((bi, 1), lambda i: (i, 0)),
            pl.BlockSpec((1, w), lambda i: (0, 0)),
        ],
        out_specs=pl.BlockSpec((bi, w), lambda i: (i, 0)),
        out_shape=jax.ShapeDtypeStruct((n, w), F32),
    )(adj123T, adj123T, Y, dis_col, bias_row)


def _dec_mid_body(h_ref, g_ref, b_ref, w4_ref, dis_ref, y4_ref):
    y4_ref[...] = (dis_ref[...] * jnp.dot(
        _bn_relu(h_ref[...], g_ref[...], b_ref[...]), w4_ref[...],
        preferred_element_type=F32)).astype(BF16)


def _simple_call(body, out_shapes, *args):
    return pl.pallas_call(body, out_shape=out_shapes)(*args)


def kernel(feat, feat_a, adj_f, adj_t, adj_i, W1, b1, W2, b2, W3, b3, W4, b4,
           bn_gamma, bn_beta, Wa1, ba1, Wa2, Wb, bb):
    n = adj_f.shape[0]
    din = W1.shape[0]
    dh = W1.shape[1]
    dout = W2.shape[1]

    gamma = bn_gamma.reshape(1, dh)
    beta = bn_beta.reshape(1, dh)
    b1r = b1.reshape(1, dh)
    b2r = b2.reshape(1, dout)
    b3r = b3.reshape(1, dh)
    b4r = b4.reshape(1, din)
    ba1r = ba1.reshape(1, -1)
    wa2r = Wa2.reshape(1, -1)
    bbr = bb.reshape(1, 1)
    wb0 = Wb[0]

    dis_f, dis_t, dis_i, rs_t, adj_fT, adj_tT, adj_iT = _degrees(
        adj_f, adj_t, adj_i)
    dis_f = dis_f.reshape(n, 1)
    dis_t = dis_t.reshape(n, 1)
    dis_i = dis_i.reshape(n, 1)

    sds = jax.ShapeDtypeStruct((n, dh), BF16)
    y1_f, y1_t, y1_a, y1_i = _simple_call(
        _prep1_body, [sds, sds, sds, sds],
        feat, feat_a, W1, dis_f, dis_t, dis_i)
    y1_ta = jnp.concatenate([y1_t, y1_a], axis=1)

    h1a_f = _gcn_mmT(adj_fT, y1_f, dis_f, b1r)
    h1a_ta = _gcn_mmT(adj_tT, y1_ta, dis_t, jnp.concatenate([b1r, b1r], axis=1))
    h1a_i = _gcn_mmT(adj_iT, y1_i, dis_i, b1r)

    sds2 = jax.ShapeDtypeStruct((n, dout), BF16)
    y2_f, y2_t, y2_a, y2_i = _simple_call(
        _mid_body, [sds2, sds2, sds2, sds2],
        h1a_f, h1a_ta, h1a_i, gamma, beta, W2, dis_f, dis_t, dis_i)
    y2_ta = jnp.concatenate([y2_t, y2_a], axis=1)

    h1_f = _gcn_mmT(adj_fT, y2_f, dis_f, b2r)
    h1_ta = _gcn_mmT(adj_tT, y2_ta, dis_t, jnp.concatenate([b2r, b2r], axis=1))
    h1_i = _gcn_mmT(adj_iT, y2_i, dis_i, b2r)
    h1_t = h1_ta[:, :dout]
    h1_a = h1_ta[:, dout:]

    g = _readout(adj_tT, h1_t, rs_t)

    col = jax.ShapeDtypeStruct((n, 1), F32)
    sc1, sc2, hiden_emb, att_f, att_t, att_i = _simple_call(
        _att_body,
        [col, col, jax.ShapeDtypeStruct((n, dout), F32), col, col, col],
        h1_f, h1_t, h1_a, h1_i, g, wb0, bbr, Wa1, ba1r, wa2r)
    ret = jnp.concatenate([sc1, sc2], axis=1)

    adj123T, cs123, rs123 = _weighted_adj(
        adj_fT, adj_tT, adj_iT, att_f, att_t, att_i)

    y3, dis_w = _simple_call(
        _dec_in_body,
        [jax.ShapeDtypeStruct((n, dh), BF16),
         jax.ShapeDtypeStruct((n, 1), F32)],
        cs123.reshape(n, 1), rs123, hiden_emb, W3)

    h2a = _sym_mm(adj123T, y3, dis_w, b3r, relu=False)

    y4 = _simple_call(
        _dec_mid_body, jax.ShapeDtypeStruct((n, din), BF16),
        h2a, gamma, beta, W4, dis_w)

    h2 = _sym_mm(adj123T, y4, dis_w, b4r, relu=True)

    return (hiden_emb, h2, ret)


# speedup vs baseline: 1.1207x; 1.1137x over previous
import functools

import jax
import jax.numpy as jnp
from jax.experimental import pallas as pl

F32 = jnp.float32
BF16 = jnp.bfloat16


def _deg_body(nsteps, f_ref, t_ref, i_ref, feat_ref, feata_ref, w1_ref,
              df_ref, dt_ref, di_ref, rs_ref, fb_ref, tb_ref, ib_ref,
              yf_ref, yta_ref, yi_ref):
    k = pl.program_id(0)
    f = f_ref[...]
    t = t_ref[...]
    i = i_ref[...]
    fb_ref[...] = f.astype(BF16)
    tb_ref[...] = t.astype(BF16)
    ib_ref[...] = i.astype(BF16)
    rs_ref[...] = jnp.sum(t, axis=1, keepdims=True)

    @pl.when(k == 0)
    def _():
        df_ref[...] = jnp.zeros_like(df_ref)
        dt_ref[...] = jnp.zeros_like(dt_ref)
        di_ref[...] = jnp.zeros_like(di_ref)

    df_ref[...] += jnp.sum(f, axis=0, keepdims=True)
    dt_ref[...] += jnp.sum(t, axis=0, keepdims=True)
    di_ref[...] += jnp.sum(i, axis=0, keepdims=True)

    @pl.when(k == nsteps - 1)
    def _():
        for r in (df_ref, dt_ref, di_ref):
            s = r[...]
            r[...] = jnp.where(s > 0, jax.lax.rsqrt(s), 0.0)
        w1 = w1_ref[...]
        p = jnp.dot(feat_ref[...], w1, preferred_element_type=F32)
        pa = jnp.dot(feata_ref[...], w1, preferred_element_type=F32)
        disf = df_ref[...].T
        dist = dt_ref[...].T
        disi = di_ref[...].T
        yf_ref[...] = (disf * p).astype(BF16)
        yta_ref[...] = jnp.concatenate(
            [(dist * p).astype(BF16), (dist * pa).astype(BF16)], axis=1)
        yi_ref[...] = (disi * p).astype(BF16)


def _degrees_prep(adj_f, adj_t, adj_i, feat, feat_a, W1, bk=128):
    n = adj_f.shape[0]
    dh = W1.shape[1]
    nk = n // bk
    return pl.pallas_call(
        functools.partial(_deg_body, nk),
        grid=(nk,),
        in_specs=[pl.BlockSpec((bk, n), lambda k: (k, 0))] * 3
        + [pl.BlockSpec((n, W1.shape[0]), lambda k: (0, 0))] * 2
        + [pl.BlockSpec(W1.shape, lambda k: (0, 0))],
        out_specs=[pl.BlockSpec((1, n), lambda k: (0, 0))] * 3
        + [pl.BlockSpec((bk, 1), lambda k: (k, 0))]
        + [pl.BlockSpec((bk, n), lambda k: (k, 0))] * 3
        + [pl.BlockSpec((n, dh), lambda k: (0, 0)),
           pl.BlockSpec((n, 2 * dh), lambda k: (0, 0)),
           pl.BlockSpec((n, dh), lambda k: (0, 0))],
        out_shape=[jax.ShapeDtypeStruct((1, n), F32)] * 3
        + [jax.ShapeDtypeStruct((n, 1), F32)]
        + [jax.ShapeDtypeStruct((n, n), BF16)] * 3
        + [jax.ShapeDtypeStruct((n, dh), BF16),
           jax.ShapeDtypeStruct((n, 2 * dh), BF16),
           jax.ShapeDtypeStruct((n, dh), BF16)],
    )(adj_f, adj_t, adj_i, feat, feat_a, W1)


def _layer_body(f_ref, t_ref, i_ref, yf_ref, yta_ref, yi_ref,
                disf_ref, dist_ref, disi_ref, bf_ref, bta_ref, bi_ref,
                of_ref, ota_ref, oi_ref):
    dims = (((0,), (0,)), ((), ()))
    of_ref[...] = (jax.lax.dot_general(f_ref[...], yf_ref[...], dims,
                                       preferred_element_type=F32)
                   * disf_ref[...] + bf_ref[...])
    ota_ref[...] = (jax.lax.dot_general(t_ref[...], yta_ref[...], dims,
                                        preferred_element_type=F32)
                    * dist_ref[...] + bta_ref[...])
    oi_ref[...] = (jax.lax.dot_general(i_ref[...], yi_ref[...], dims,
                                       preferred_element_type=F32)
                   * disi_ref[...] + bi_ref[...])


def _enc_layer(fb, tb, ib, y_f, y_ta, y_i, dis_f, dis_t, dis_i,
               b_f, b_ta, b_i, bi=512):
    n = fb.shape[0]
    wf = y_f.shape[1]
    wt = y_ta.shape[1]
    return pl.pallas_call(
        _layer_body,
        grid=(n // bi,),
        in_specs=[
            pl.BlockSpec((n, bi), lambda i: (0, i)),
            pl.BlockSpec((n, bi), lambda i: (0, i)),
            pl.BlockSpec((n, bi), lambda i: (0, i)),
            pl.BlockSpec((n, wf), lambda i: (0, 0)),
            pl.BlockSpec((n, wt), lambda i: (0, 0)),
            pl.BlockSpec((n, wf), lambda i: (0, 0)),
            pl.BlockSpec((bi, 1), lambda i: (i, 0)),
            pl.BlockSpec((bi, 1), lambda i: (i, 0)),
            pl.BlockSpec((bi, 1), lambda i: (i, 0)),
            pl.BlockSpec((1, wf), lambda i: (0, 0)),
            pl.BlockSpec((1, wt), lambda i: (0, 0)),
            pl.BlockSpec((1, wf), lambda i: (0, 0)),
        ],
        out_specs=[
            pl.BlockSpec((bi, wf), lambda i: (i, 0)),
            pl.BlockSpec((bi, wt), lambda i: (i, 0)),
            pl.BlockSpec((bi, wf), lambda i: (i, 0)),
        ],
        out_shape=[
            jax.ShapeDtypeStruct((n, wf), F32),
            jax.ShapeDtypeStruct((n, wt), F32),
            jax.ShapeDtypeStruct((n, wf), F32),
        ],
    )(fb, tb, ib, y_f, y_ta, y_i, dis_f, dis_t, dis_i, b_f, b_ta, b_i)


def _bn_relu(h, gamma, beta):
    m = jnp.mean(h, axis=0, keepdims=True)
    c = h - m
    v = jnp.mean(c * c, axis=0, keepdims=True)
    return jnp.maximum(c * jax.lax.rsqrt(v + 1e-5) * gamma + beta, 0.0)


def _mid_body(hf_ref, hta_ref, hi_ref, g_ref, b_ref, w2_ref,
              df_ref, dt_ref, di_ref, yf_ref, yta_ref, yi_ref):
    g = g_ref[...]
    b = b_ref[...]
    w2 = w2_ref[...]
    dh = w2.shape[0]

    def stage(h, dis):
        return (dis * jnp.dot(_bn_relu(h, g, b), w2,
                              preferred_element_type=F32)).astype(BF16)

    yf_ref[...] = stage(hf_ref[...], df_ref[...])
    yta_ref[...] = jnp.concatenate(
        [stage(hta_ref[:, :dh], dt_ref[...]),
         stage(hta_ref[:, dh:], dt_ref[...])], axis=1)
    yi_ref[...] = stage(hi_ref[...], di_ref[...])


def _ratt_body(tb_ref, h1t_full_ref, rs_ref, hf_ref, ht_ref, ha_ref, hi_ref,
               wb_ref, bb_ref, wa1_ref, ba1_ref, wa2_ref,
               sc1_ref, sc2_ref, hid_ref, bf_ref, bt_ref, bi_ref):
    acc = jnp.dot(tb_ref[...], h1t_full_ref[...].astype(BF16),
                  preferred_element_type=F32)
    gg = acc / rs_ref[...]
    nrm = jnp.sqrt(jnp.sum(gg * gg, axis=1, keepdims=True))
    gg = gg / jnp.maximum(nrm, 1e-12)
    g = jax.nn.sigmoid(gg)

    hf = hf_ref[...]
    ht = ht_ref[...]
    ha = ha_ref[...]
    hi = hi_ref[...]
    wb = wb_ref[...]
    u_t = jnp.dot(ht, wb, preferred_element_type=F32)
    u_a = jnp.dot(ha, wb, preferred_element_type=F32)
    sc1_ref[...] = jnp.sum(u_t * g, axis=1, keepdims=True) + bb_ref[...]
    sc2_ref[...] = jnp.sum(u_a * g, axis=1, keepdims=True) + bb_ref[...]

    wa1 = wa1_ref[...]
    ba1 = ba1_ref[...]
    wa2 = wa2_ref[...]

    def score(h):
        e = jnp.tanh(jnp.dot(h, wa1, preferred_element_type=F32) + ba1)
        return jnp.sum(e * wa2, axis=1, keepdims=True)

    ef = score(hf)
    et = score(ht)
    ei = score(hi)
    mx = jnp.maximum(jnp.maximum(ef, et), ei)
    xf = jnp.exp(ef - mx)
    xt = jnp.exp(et - mx)
    xi = jnp.exp(ei - mx)
    s = xf + xt + xi
    bf = xf / s
    bt = xt / s
    bi = xi / s
    bf_ref[...] = bf
    bt_ref[...] = bt
    bi_ref[...] = bi
    hid_ref[...] = bf * hf + bt * ht + bi * hi


def _readout_att(tb, h1_t, rs_t, h1_f, h1_a, h1_i, wb0, bbr, Wa1, ba1r, wa2r,
                 bi=1024):
    n = tb.shape[0]
    w = h1_t.shape[1]
    col = jax.ShapeDtypeStruct((n, 1), F32)
    blk = lambda i: (i, 0)
    return pl.pallas_call(
        _ratt_body,
        grid=(n // bi,),
        in_specs=[
            pl.BlockSpec((bi, n), blk),
            pl.BlockSpec((n, w), lambda i: (0, 0)),
            pl.BlockSpec((bi, 1), blk),
            pl.BlockSpec((bi, w), blk),
            pl.BlockSpec((bi, w), blk),
            pl.BlockSpec((bi, w), blk),
            pl.BlockSpec((bi, w), blk),
            pl.BlockSpec(wb0.shape, lambda i: (0, 0)),
            pl.BlockSpec((1, 1), lambda i: (0, 0)),
            pl.BlockSpec(Wa1.shape, lambda i: (0, 0)),
            pl.BlockSpec(ba1r.shape, lambda i: (0, 0)),
            pl.BlockSpec(wa2r.shape, lambda i: (0, 0)),
        ],
        out_specs=[
            pl.BlockSpec((bi, 1), blk),
            pl.BlockSpec((bi, 1), blk),
            pl.BlockSpec((bi, w), blk),
            pl.BlockSpec((bi, 1), blk),
            pl.BlockSpec((bi, 1), blk),
            pl.BlockSpec((bi, 1), blk),
        ],
        out_shape=[col, col, jax.ShapeDtypeStruct((n, w), F32),
                   col, col, col],
    )(tb, h1_t, rs_t, h1_f, h1_t, h1_a, h1_i, wb0, bbr, Wa1, ba1r, wa2r)


def _wadj_body(nsteps, bk, f_ref, t_ref, i_ref, af_ref, at_ref, ai_ref,
               hid_ref, w3_ref, a_ref, cs_ref, rs_ref, y3_ref, dis_ref):
    k = pl.program_id(0)
    blk = (af_ref[...] * f_ref[...].astype(F32)
           + at_ref[...] * t_ref[...].astype(F32)
           + ai_ref[...] * i_ref[...].astype(F32))
    a_ref[...] = blk.astype(BF16)
    rs_ref[pl.ds(k * bk, bk), :] = jnp.sum(blk, axis=1, keepdims=True)

    @pl.when(k == 0)
    def _():
        cs_ref[...] = jnp.zeros_like(cs_ref)

    cs_ref[...] += jnp.sum(blk, axis=0, keepdims=True)

    @pl.when(k == nsteps - 1)
    def _():
        dw = (cs_ref[...].T + rs_ref[...]) / 3.0
        dis = jnp.where(dw > 0, jax.lax.rsqrt(dw), 0.0)
        dis_ref[...] = dis
        y3_ref[...] = (dis * jnp.dot(hid_ref[...], w3_ref[...],
                                     preferred_element_type=F32)).astype(BF16)


def _weighted_adj(fb, tb, ib, att_f, att_t, att_i, hiden, W3, bk=256):
    n = fb.shape[0]
    dh = W3.shape[1]
    nk = n // bk
    return pl.pallas_call(
        functools.partial(_wadj_body, nk, bk),
        grid=(nk,),
        in_specs=[pl.BlockSpec((bk, n), lambda k: (k, 0))] * 3
        + [pl.BlockSpec((bk, 1), lambda k: (k, 0))] * 3
        + [pl.BlockSpec(hiden.shape, lambda k: (0, 0)),
           pl.BlockSpec(W3.shape, lambda k: (0, 0))],
        out_specs=[
            pl.BlockSpec((bk, n), lambda k: (k, 0)),
            pl.BlockSpec((1, n), lambda k: (0, 0)),
            pl.BlockSpec((n, 1), lambda k: (0, 0)),
            pl.BlockSpec((n, dh), lambda k: (0, 0)),
            pl.BlockSpec((n, 1), lambda k: (0, 0)),
        ],
        out_shape=[
            jax.ShapeDtypeStruct((n, n), BF16),
            jax.ShapeDtypeStruct((1, n), F32),
            jax.ShapeDtypeStruct((n, 1), F32),
            jax.ShapeDtypeStruct((n, dh), BF16),
            jax.ShapeDtypeStruct((n, 1), F32),
        ],
    )(fb, tb, ib, att_f, att_t, att_i, hiden, W3)


def _sym_body(relu, a1_ref, a2_ref, y_ref, dis_ref, b_ref, o_ref):
    y = y_ref[...]
    acc = jnp.dot(a1_ref[...], y, preferred_element_type=F32)
    acc += jax.lax.dot_general(
        a2_ref[...], y, (((0,), (0,)), ((), ())), preferred_element_type=F32)
    r = dis_ref[...] * (acc * (1.0 / 3.0)) + b_ref[...]
    if relu:
        r = jnp.maximum(r, 0.0)
    o_ref[...] = r


def _sym_mm(adj123, Y, dis_col, bias_row, relu, bi=1024):
    n = adj123.shape[0]
    w = Y.shape[1]
    return pl.pallas_call(
        functools.partial(_sym_body, relu),
        grid=(n // bi,),
        in_specs=[
            pl.BlockSpec((bi, n), lambda i: (i, 0)),
            pl.BlockSpec((n, bi), lambda i: (0, i)),
            pl.BlockSpec((n, w), lambda i: (0, 0)),
            pl.BlockSpec((bi, 1), lambda i: (i, 0)),
            pl.BlockSpec((1, w), lambda i: (0, 0)),
        ],
        out_specs=pl.BlockSpec((bi, w), lambda i: (i, 0)),
        out_shape=jax.ShapeDtypeStruct((n, w), F32),
    )(adj123, adj123, Y, dis_col, bias_row)


def _dec_mid_body(h_ref, g_ref, b_ref, w4_ref, dis_ref, y4_ref):
    y4_ref[...] = (dis_ref[...] * jnp.dot(
        _bn_relu(h_ref[...], g_ref[...], b_ref[...]), w4_ref[...],
        preferred_element_type=F32)).astype(BF16)


def _simple_call(body, out_shapes, *args):
    return pl.pallas_call(body, out_shape=out_shapes)(*args)


def kernel(feat, feat_a, adj_f, adj_t, adj_i, W1, b1, W2, b2, W3, b3, W4, b4,
           bn_gamma, bn_beta, Wa1, ba1, Wa2, Wb, bb):
    n = adj_f.shape[0]
    din = W1.shape[0]
    dh = W1.shape[1]
    dout = W2.shape[1]

    gamma = bn_gamma.reshape(1, dh)
    beta = bn_beta.reshape(1, dh)
    b1r = b1.reshape(1, dh)
    b2r = b2.reshape(1, dout)
    b3r = b3.reshape(1, dh)
    b4r = b4.reshape(1, din)
    ba1r = ba1.reshape(1, -1)
    wa2r = Wa2.reshape(1, -1)
    bbr = bb.reshape(1, 1)
    wb0 = Wb[0]

    (dis_f, dis_t, dis_i, rs_t, fb, tb, ib,
     y1_f, y1_ta, y1_i) = _degrees_prep(adj_f, adj_t, adj_i, feat, feat_a, W1)
    dis_f = dis_f.reshape(n, 1)
    dis_t = dis_t.reshape(n, 1)
    dis_i = dis_i.reshape(n, 1)

    b1ta = jnp.concatenate([b1r, b1r], axis=1)
    h1a_f, h1a_ta, h1a_i = _enc_layer(
        fb, tb, ib, y1_f, y1_ta, y1_i, dis_f, dis_t, dis_i, b1r, b1ta, b1r)

    sds2 = jax.ShapeDtypeStruct((n, dout), BF16)
    y2_f, y2_ta, y2_i = _simple_call(
        _mid_body,
        [sds2, jax.ShapeDtypeStruct((n, 2 * dout), BF16), sds2],
        h1a_f, h1a_ta, h1a_i, gamma, beta, W2, dis_f, dis_t, dis_i)

    b2ta = jnp.concatenate([b2r, b2r], axis=1)
    h1_f, h1_ta, h1_i = _enc_layer(
        fb, tb, ib, y2_f, y2_ta, y2_i, dis_f, dis_t, dis_i, b2r, b2ta, b2r)
    h1_t = h1_ta[:, :dout]
    h1_a = h1_ta[:, dout:]

    sc1, sc2, hiden_emb, att_f, att_t, att_i = _readout_att(
        tb, h1_t, rs_t, h1_f, h1_a, h1_i, wb0, bbr, Wa1, ba1r, wa2r)
    ret = jnp.concatenate([sc1, sc2], axis=1)

    adj123, _cs, _rs, y3, dis_w = _weighted_adj(
        fb, tb, ib, att_f, att_t, att_i, hiden_emb, W3)

    h2a = _sym_mm(adj123, y3, dis_w, b3r, relu=False)

    y4 = _simple_call(
        _dec_mid_body, jax.ShapeDtypeStruct((n, din), BF16),
        h2a, gamma, beta, W4, dis_w)

    h2 = _sym_mm(adj123, y4, dis_w, b4r, relu=True)

    return (hiden_emb, h2, ret)
